# baseline (speedup 1.0000x reference)
import jax
import jax.numpy as jnp
from jax.experimental import pallas as pl
from jax.experimental.pallas import tpu as pltpu

LN_EPS = 1e-5
NORM_EPS = 1e-12


def _table_kernel(e_ref, w1_ref, b1_ref, w2_ref, b2_ref, g_ref, bb_ref,
                  wkp_ref, out_ref):
    e = e_ref[...]
    t = jnp.maximum(
        jnp.dot(e, w1_ref[...], preferred_element_type=jnp.float32)
        + b1_ref[...], 0.0)
    ff = jnp.dot(t, w2_ref[...], preferred_element_type=jnp.float32) + b2_ref[...]
    x = e + ff
    mu = jnp.mean(x, axis=-1, keepdims=True)
    xc = x - mu
    var = jnp.mean(xc * xc, axis=-1, keepdims=True)
    xn = xc * jax.lax.rsqrt(var + LN_EPS) * g_ref[...] + bb_ref[...]
    kp = jnp.dot(xn, wkp_ref[...], preferred_element_type=jnp.float32)
    nrm = jnp.sqrt(jnp.sum(kp * kp, axis=-1, keepdims=True))
    kn = kp * (1.0 / jnp.maximum(nrm, NORM_EPS))
    kpu = jax.lax.bitcast_convert_type(kp.astype(jnp.bfloat16),
                                       jnp.uint16).astype(jnp.uint32)
    knu = jax.lax.bitcast_convert_type(kn.astype(jnp.bfloat16),
                                       jnp.uint16).astype(jnp.uint32)
    packed = (knu << 16) | kpu
    out_ref[...] = jnp.concatenate([packed, packed], axis=-1)


def _scan_kernel(idx_ref, tab_hbm, read_ref, tab_ref, pk_s, sem):
    B, H = read_ref.shape
    L = idx_ref.shape[0] // B
    K = 4
    n_groups = (L - K) // K
    cp = pltpu.make_async_copy(tab_hbm, tab_ref, sem)
    cp.start()
    cp.wait()

    KB = K * B

    def gather_block(base_step, n, buf):
        off0 = base_step * B
        boff = buf * KB
        for i in range(n):
            for b in range(B):
                i2 = pl.multiple_of(idx_ref[off0 + (i * B + b)], 2)
                slab = tab_ref[pl.ds(i2, 2), :]
                pk_s[pl.ds(boff + i * B + b, 1), :] = slab[0:1, :]

    def group_compute(KP4, KN4, u, read):
        kns = [KN4[i * B:(i + 1) * B, :] for i in range(K)]
        a = [jnp.sum(kns[i] * u, axis=-1, keepdims=True) for i in range(K)]
        d = {}
        for i in range(1, K):
            for j2 in range(i):
                d[(i, j2)] = jnp.sum(kns[i] * kns[j2], axis=-1, keepdims=True)
        c = [a[0]]
        for i in range(1, K):
            ci = a[i]
            for j2 in range(i):
                ci = ci - d[(i, j2)] * c[j2]
            c.append(ci)
        return c

    def unpack_at(base, n):
        s = pk_s[pl.ds(base, n), :]
        kp = pltpu.bitcast(s << 16, jnp.float32)
        kn = pltpu.bitcast(s & jnp.uint32(0xFFFF0000), jnp.float32)
        return kp, kn

    def deferred_group(cur_base, prev_base, u_base, read, c_prev):
        KPc, KNc = unpack_at(cur_base, KB)
        KPp, KNp = unpack_at(prev_base, KB)
        knc = [KNc[i * B:(i + 1) * B, :] for i in range(K)]
        knp = [KNp[i * B:(i + 1) * B, :] for i in range(K)]
        kpp = [KPp[i * B:(i + 1) * B, :] for i in range(K)]
        a_raw = [jnp.sum(knc[i] * u_base, axis=-1, keepdims=True)
                 for i in range(K)]
        cg = [[jnp.sum(knc[i] * knp[j2], axis=-1, keepdims=True)
               for j2 in range(K)] for i in range(K)]
        d = {}
        for i in range(1, K):
            for j2 in range(i):
                d[(i, j2)] = jnp.sum(knc[i] * knc[j2], axis=-1, keepdims=True)
        c = []
        for i in range(K):
            ci = a_raw[i]
            for j2 in range(K):
                ci = ci - cg[i][j2] * c_prev[j2]
            for j2 in range(i):
                ci = ci - d[(i, j2)] * c[j2]
            c.append(ci)
        upd = c_prev[0] * knp[0]
        acc = c_prev[0] * kpp[0]
        for i in range(1, K):
            upd = upd + c_prev[i] * knp[i]
            acc = acc + c_prev[i] * kpp[i]
        return u_base - upd, read + acc, c

    def apply_group(base, u_base, read, c_prev):
        KPp, KNp = unpack_at(base, KB)
        knp = [KNp[i * B:(i + 1) * B, :] for i in range(K)]
        kpp = [KPp[i * B:(i + 1) * B, :] for i in range(K)]
        upd = c_prev[0] * knp[0]
        acc = c_prev[0] * kpp[0]
        for i in range(1, K):
            upd = upd + c_prev[i] * knp[i]
            acc = acc + c_prev[i] * kpp[i]
        return u_base - upd, read + acc

    gather_block(0, 1, 0)
    u, _ = unpack_at(0, B)
    read = jnp.zeros((B, H), jnp.float32)

    for j in range(1, K):
        gather_block(j, 1, 0)
        kp, kn = unpack_at(0, B)
        c = jnp.sum(kn * u, axis=-1, keepdims=True)
        read = read + c * kp
        u = u - c * kn

    gather_block(K, K, 0)
    gather_block(2 * K, K, 1)
    gather_block(3 * K, K, 2)
    _, KN0 = unpack_at(0, KB)
    c_prev = group_compute(None, KN0, u, read)
    n_bodies = (n_groups - 3) // 3

    def body(h, carry):
        u_base, read, c1, c2, c3, c4 = carry
        cp_ = [c1, c2, c3, c4]
        u_base, read, cp_ = deferred_group(KB, 0, u_base, read, cp_)
        gather_block(K * (3 * h + 4), K, 0)
        u_base, read, cp_ = deferred_group(2 * KB, KB, u_base, read, cp_)
        gather_block(K * (3 * h + 5), K, 1)
        u_base, read, cp_ = deferred_group(0, 2 * KB, u_base, read, cp_)
        gather_block(K * (3 * h + 6), K, 2)
        return (u_base, read, cp_[0], cp_[1], cp_[2], cp_[3])

    carry = jax.lax.fori_loop(0, n_bodies, body,
                              (u, read, c_prev[0], c_prev[1], c_prev[2],
                               c_prev[3]))
    u, read = carry[0], carry[1]
    c_prev = list(carry[2:])
    u, read, c_prev = deferred_group(KB, 0, u, read, c_prev)
    u, read, c_prev = deferred_group(2 * KB, KB, u, read, c_prev)
    _, read = apply_group(2 * KB, u, read, c_prev)
    read_ref[...] = read
    read_ref[...] = read


def _out_kernel(read_ref, wrp_ref, brp_ref, wout_ref, bout_ref, o_ref):
    r2 = jnp.dot(read_ref[...], wrp_ref[...],
                 preferred_element_type=jnp.float32) + brp_ref[...]
    o_ref[...] = jnp.dot(r2, wout_ref[...],
                         preferred_element_type=jnp.float32) + bout_ref[...]


def kernel(seq, embed, W1, b1, W2, b2, ln_g, ln_b, Wkp, Wrp, brp, Wout, bout):
    V, H = embed.shape
    B, L = seq.shape
    H2 = W1.shape[1]

    tile_v = min(800, V)
    tab = pl.pallas_call(
        _table_kernel,
        out_shape=jax.ShapeDtypeStruct((V, 2 * H), jnp.uint32),
        grid=(V // tile_v,),
        in_specs=[
            pl.BlockSpec((tile_v, H), lambda i: (i, 0)),
            pl.BlockSpec((H, H2), lambda i: (0, 0)),
            pl.BlockSpec((1, H2), lambda i: (0, 0)),
            pl.BlockSpec((H2, H), lambda i: (0, 0)),
            pl.BlockSpec((1, H), lambda i: (0, 0)),
            pl.BlockSpec((1, H), lambda i: (0, 0)),
            pl.BlockSpec((1, H), lambda i: (0, 0)),
            pl.BlockSpec((H, H), lambda i: (0, 0)),
        ],
        out_specs=pl.BlockSpec((tile_v, 2 * H), lambda i: (i, 0)),
        compiler_params=pltpu.CompilerParams(
            dimension_semantics=("parallel",),
        ),
        name="kp_kn_table",
    )(embed, W1, b1.reshape(1, H2), W2, b2.reshape(1, H), ln_g.reshape(1, H),
      ln_b.reshape(1, H), Wkp)
    tab2 = tab.reshape(2 * V, H)

    flat2 = (seq[:, ::-1].T.astype(jnp.int32) * 2).reshape(-1)

    read = pl.pallas_call(
        _scan_kernel,
        out_shape=jax.ShapeDtypeStruct((B, H), jnp.float32),
        grid_spec=pltpu.PrefetchScalarGridSpec(
            num_scalar_prefetch=1,
            grid=(1,),
            in_specs=[pl.BlockSpec(memory_space=pl.ANY)],
            out_specs=pl.BlockSpec((B, H), lambda i, *_: (0, 0)),
            scratch_shapes=[
                pltpu.VMEM((2 * V, H), jnp.uint32),
                pltpu.VMEM((12 * B, H), jnp.uint32),
                pltpu.SemaphoreType.DMA,
            ],
        ),
        compiler_params=pltpu.CompilerParams(
            dimension_semantics=("arbitrary",),
            vmem_limit_bytes=52 * 1024 * 1024,
        ),
        name="delta_scan",
    )(flat2, tab2)

    tile_o = min(3200, V)
    out = pl.pallas_call(
        _out_kernel,
        out_shape=jax.ShapeDtypeStruct((B, V), jnp.float32),
        grid=(V // tile_o,),
        in_specs=[
            pl.BlockSpec((B, H), lambda i: (0, 0)),
            pl.BlockSpec((H, H), lambda i: (0, 0)),
            pl.BlockSpec((1, H), lambda i: (0, 0)),
            pl.BlockSpec((H, tile_o), lambda i: (0, i)),
            pl.BlockSpec((1, tile_o), lambda i: (0, i)),
        ],
        out_specs=pl.BlockSpec((B, tile_o), lambda i: (0, i)),
        compiler_params=pltpu.CompilerParams(
            dimension_semantics=("parallel",),
        ),
        name="readout_proj",
    )(read, Wrp, brp.reshape(1, H), Wout, bout.reshape(1, V))
    return out


# speedup vs baseline: 1.0829x; 1.0829x over previous
import jax
import jax.numpy as jnp
from jax.experimental import pallas as pl
from jax.experimental.pallas import tpu as pltpu

LN_EPS = 1e-5
NORM_EPS = 1e-12


def _table_kernel(e_ref, w1_ref, b1_ref, w2_ref, b2_ref, g_ref, bb_ref,
                  wkp_ref, out_ref):
    e = e_ref[...]
    t = jnp.maximum(
        jnp.dot(e, w1_ref[...], preferred_element_type=jnp.float32)
        + b1_ref[...], 0.0)
    ff = jnp.dot(t, w2_ref[...], preferred_element_type=jnp.float32) + b2_ref[...]
    x = e + ff
    mu = jnp.mean(x, axis=-1, keepdims=True)
    xc = x - mu
    var = jnp.mean(xc * xc, axis=-1, keepdims=True)
    xn = xc * jax.lax.rsqrt(var + LN_EPS) * g_ref[...] + bb_ref[...]
    kp = jnp.dot(xn, wkp_ref[...], preferred_element_type=jnp.float32)
    nrm = jnp.sqrt(jnp.sum(kp * kp, axis=-1, keepdims=True))
    kn = kp * (1.0 / jnp.maximum(nrm, NORM_EPS))
    kpu = jax.lax.bitcast_convert_type(kp.astype(jnp.bfloat16),
                                       jnp.uint16).astype(jnp.uint32)
    knu = jax.lax.bitcast_convert_type(kn.astype(jnp.bfloat16),
                                       jnp.uint16).astype(jnp.uint32)
    packed = (knu << 16) | kpu
    out_ref[...] = jnp.concatenate([packed, packed], axis=-1)


def _scan_kernel(idx_ref, tab_hbm, read_ref, tab_ref, pk_s, sem):
    B, H = read_ref.shape
    L = idx_ref.shape[0] // B
    K = 4
    n_groups = (L - K) // K
    cp = pltpu.make_async_copy(tab_hbm, tab_ref, sem)
    cp.start()
    cp.wait()

    KB = K * B

    def gather_block(base_step, n, buf):
        off0 = base_step * B
        boff = buf * KB
        for i in range(n):
            for b in range(B):
                i2 = pl.multiple_of(idx_ref[off0 + (i * B + b)], 2)
                slab = tab_ref[pl.ds(i2, 2), :]
                pk_s[pl.ds(boff + i * B + b, 1), :] = slab[0:1, :]

    def group_compute(KP4, KN4, u, read):
        kns = [KN4[i * B:(i + 1) * B, :] for i in range(K)]
        kps = [KP4[i * B:(i + 1) * B, :] for i in range(K)]
        a = [jnp.sum(kns[i] * u, axis=-1, keepdims=True) for i in range(K)]
        d = {}
        for i in range(1, K):
            for j2 in range(i):
                d[(i, j2)] = jnp.sum(kns[i] * kns[j2], axis=-1, keepdims=True)
        c = [a[0]]
        for i in range(1, K):
            ci = a[i]
            for j2 in range(i):
                ci = ci - d[(i, j2)] * c[j2]
            c.append(ci)
        upd = c[0] * kns[0]
        acc = c[0] * kps[0]
        for i in range(1, K):
            upd = upd + c[i] * kns[i]
            acc = acc + c[i] * kps[i]
        return u - upd, read + acc

    def unpack(n):
        s = pk_s[pl.ds(0, n), :]
        kp = pltpu.bitcast(s << 16, jnp.float32)
        kn = pltpu.bitcast(s & jnp.uint32(0xFFFF0000), jnp.float32)
        return kp, kn

    gather_block(0, 1, 0)
    u, _ = unpack(B)
    read = jnp.zeros((B, H), jnp.float32)

    for j in range(1, K):
        gather_block(j, 1, 0)
        kp, kn = unpack(B)
        c = jnp.sum(kn * u, axis=-1, keepdims=True)
        read = read + c * kp
        u = u - c * kn

    gather_block(K, K, 0)

    def body(g, carry):
        u, read = carry
        KP4, KN4 = unpack(KB)
        gather_block(2 * K + K * g, K, 0)
        u, read = group_compute(KP4, KN4, u, read)
        return (u, read)

    u, read = jax.lax.fori_loop(0, n_groups - 1, body, (u, read))
    KP4, KN4 = unpack(KB)
    _, read = group_compute(KP4, KN4, u, read)
    read_ref[...] = read


def _out_kernel(read_ref, wrp_ref, brp_ref, wout_ref, bout_ref, o_ref):
    r2 = jnp.dot(read_ref[...], wrp_ref[...],
                 preferred_element_type=jnp.float32) + brp_ref[...]
    o_ref[...] = jnp.dot(r2, wout_ref[...],
                         preferred_element_type=jnp.float32) + bout_ref[...]


def kernel(seq, embed, W1, b1, W2, b2, ln_g, ln_b, Wkp, Wrp, brp, Wout, bout):
    V, H = embed.shape
    B, L = seq.shape
    H2 = W1.shape[1]

    tile_v = min(800, V)
    tab = pl.pallas_call(
        _table_kernel,
        out_shape=jax.ShapeDtypeStruct((V, 2 * H), jnp.uint32),
        grid=(V // tile_v,),
        in_specs=[
            pl.BlockSpec((tile_v, H), lambda i: (i, 0)),
            pl.BlockSpec((H, H2), lambda i: (0, 0)),
            pl.BlockSpec((1, H2), lambda i: (0, 0)),
            pl.BlockSpec((H2, H), lambda i: (0, 0)),
            pl.BlockSpec((1, H), lambda i: (0, 0)),
            pl.BlockSpec((1, H), lambda i: (0, 0)),
            pl.BlockSpec((1, H), lambda i: (0, 0)),
            pl.BlockSpec((H, H), lambda i: (0, 0)),
        ],
        out_specs=pl.BlockSpec((tile_v, 2 * H), lambda i: (i, 0)),
        compiler_params=pltpu.CompilerParams(
            dimension_semantics=("parallel",),
        ),
        name="kp_kn_table",
    )(embed, W1, b1.reshape(1, H2), W2, b2.reshape(1, H), ln_g.reshape(1, H),
      ln_b.reshape(1, H), Wkp)
    tab2 = tab.reshape(2 * V, H)

    flat2 = (seq[:, ::-1].T.astype(jnp.int32) * 2).reshape(-1)

    read = pl.pallas_call(
        _scan_kernel,
        out_shape=jax.ShapeDtypeStruct((B, H), jnp.float32),
        grid_spec=pltpu.PrefetchScalarGridSpec(
            num_scalar_prefetch=1,
            grid=(1,),
            in_specs=[pl.BlockSpec(memory_space=pl.ANY)],
            out_specs=pl.BlockSpec((B, H), lambda i, *_: (0, 0)),
            scratch_shapes=[
                pltpu.VMEM((2 * V, H), jnp.uint32),
                pltpu.VMEM((4 * B, H), jnp.uint32),
                pltpu.SemaphoreType.DMA,
            ],
        ),
        compiler_params=pltpu.CompilerParams(
            dimension_semantics=("arbitrary",),
            vmem_limit_bytes=52 * 1024 * 1024,
        ),
        name="delta_scan",
    )(flat2, tab2)

    tile_o = min(3200, V)
    out = pl.pallas_call(
        _out_kernel,
        out_shape=jax.ShapeDtypeStruct((B, V), jnp.float32),
        grid=(V // tile_o,),
        in_specs=[
            pl.BlockSpec((B, H), lambda i: (0, 0)),
            pl.BlockSpec((H, H), lambda i: (0, 0)),
            pl.BlockSpec((1, H), lambda i: (0, 0)),
            pl.BlockSpec((H, tile_o), lambda i: (0, i)),
            pl.BlockSpec((1, tile_o), lambda i: (0, i)),
        ],
        out_specs=pl.BlockSpec((B, tile_o), lambda i: (0, i)),
        compiler_params=pltpu.CompilerParams(
            dimension_semantics=("parallel",),
        ),
        name="readout_proj",
    )(read, Wrp, brp.reshape(1, H), Wout, bout.reshape(1, V))
    return out
